# baseline (speedup 1.0000x reference)
import functools

import jax
import jax.numpy as jnp
from jax import lax
from jax.experimental import pallas as pl
from jax.experimental.pallas import tpu as pltpu
from jax.experimental.pallas import tpu_sc as plsc

_NW = 32
_CH = 64


def _sc_body(pos_hbm, hid_hbm, emb_hbm, out_hbm, pos_v, buf, ctxbuf, ftile, sem):
    B, S, D = hid_hbm.shape
    NCT = emb_hbm.shape[1]
    nseg = _NW // B
    body = S - NCT
    per_seg = (body + nseg - 1) // nseg

    c = lax.axis_index("c")
    s = lax.axis_index("s")
    wid = s * 2 + c
    b = wid // nseg
    seg = wid % nseg

    def shift_down(n):
        def row(r, carry):
            for col in range(D // 16):
                buf[r, pl.ds(col * 16, 16)] = buf[r + NCT, pl.ds(col * 16, 16)]
            return carry

        lax.fori_loop(0, n, row, 0)

    for j in range(per_seg // _CH):
        last = j == per_seg // _CH - 1
        R = 2 * NCT + seg * per_seg + j * _CH

        @pl.when(jnp.logical_or(seg < nseg - 1, jnp.bool_(not last)))
        def _():
            fetch = pltpu.make_async_copy(
                hid_hbm.at[b, pl.ds(R - 2 * NCT, _CH + 8)], buf, sem
            )
            fetch.start()
            fetch.wait()
            shift_down(_CH)
            put = pltpu.make_async_copy(
                buf.at[pl.ds(0, _CH)], out_hbm.at[b, pl.ds(R, _CH)], sem
            )
            put.start()
            put.wait()

        if last:

            @pl.when(seg == nseg - 1)
            def _():
                fetch = pltpu.make_async_copy(
                    hid_hbm.at[b, pl.ds(R - 2 * NCT, _CH)],
                    buf.at[pl.ds(0, _CH)],
                    sem,
                )
                fetch.start()
                fetch.wait()
                shift_down(_CH - NCT)
                put = pltpu.make_async_copy(
                    buf.at[pl.ds(0, _CH - 2 * NCT)],
                    out_hbm.at[b, pl.ds(R, _CH - 2 * NCT)],
                    sem,
                )
                put.start()
                tail = pltpu.make_async_copy(
                    buf.at[pl.ds(_CH - 2 * NCT, NCT)],
                    out_hbm.at[b, pl.ds(R + _CH - 2 * NCT, NCT)],
                    sem,
                )
                tail.start()
                put.wait()
                tail.wait()

    @pl.when(seg == 0)
    def _():
        cp = pltpu.make_async_copy(pos_hbm, pos_v, sem)
        cp.start()
        cp.wait()
        cp = pltpu.make_async_copy(emb_hbm.at[pos_v], ctxbuf, sem)
        cp.start()
        cp.wait()
        cp = pltpu.make_async_copy(
            hid_hbm.at[b, pl.ds(0, 2 * NCT)], buf.at[pl.ds(0, 2 * NCT)], sem
        )
        cp.start()
        cp.wait()
        for r in range(NCT):
            for col in range(D // 16):
                sl = pl.ds(col * 16, 16)
                ftile[r, sl] = ctxbuf[0, r, sl]
                ftile[NCT + r, sl] = buf[r, sl]
        cp = pltpu.make_async_copy(ftile, out_hbm.at[b, pl.ds(0, 2 * NCT)], sem)
        cp.start()
        cp.wait()


def _mask_body(mask_ref, mask_out_ref):
    nct = mask_out_ref.shape[2] - mask_ref.shape[2]
    mask_out_ref[0, 0, :nct] = jnp.ones((nct,), mask_out_ref.dtype)
    mask_out_ref[0, 0, nct:] = mask_ref[0, 0]


def kernel(hidden_states, attention_mask, embeddings, position):
    B, S, D = hidden_states.shape
    _, NCT, _ = embeddings.shape
    pos = jnp.asarray(position, jnp.int32).reshape((1,))

    mesh = plsc.VectorSubcoreMesh(core_axis_name="c", subcore_axis_name="s")
    sc_kernel = functools.partial(
        pl.kernel,
        mesh=mesh,
        out_type=jax.ShapeDtypeStruct((B, NCT + S, D), hidden_states.dtype),
        scratch_types=[
            pltpu.VMEM((1,), jnp.int32),
            pltpu.VMEM((_CH + 8, D), hidden_states.dtype),
            pltpu.VMEM((1, NCT, D), hidden_states.dtype),
            pltpu.VMEM((2 * NCT, D), hidden_states.dtype),
            pltpu.SemaphoreType.DMA,
        ],
    )(_sc_body)
    out_hid = sc_kernel(pos, hidden_states, embeddings)

    mask3 = attention_mask.reshape(B, 1, S)
    out_mask = pl.pallas_call(
        _mask_body,
        grid=(B,),
        in_specs=[pl.BlockSpec((1, 1, S), lambda b: (b, 0, 0))],
        out_specs=pl.BlockSpec((1, 1, NCT + S), lambda b: (b, 0, 0)),
        out_shape=jax.ShapeDtypeStruct((B, 1, NCT + S), attention_mask.dtype),
    )(mask3)
    return (out_hid, out_mask.reshape(B, NCT + S))


# speedup vs baseline: 2.0959x; 2.0959x over previous
import jax
import jax.numpy as jnp
from jax.experimental import pallas as pl
from jax.experimental.pallas import tpu as pltpu

D_BLK = 512


def _body(pos_ref, hid_ref, mask_ref, emb_ref, out_ref, mask_out_ref):
    nct = emb_ref.shape[1]
    out_ref[0, :nct, :] = emb_ref[0]
    out_ref[0, nct:, :] = hid_ref[0]
    d = pl.program_id(1)

    @pl.when(d == 0)
    def _():
        mask_out_ref[0, 0, :nct] = jnp.ones((nct,), mask_out_ref.dtype)
        mask_out_ref[0, 0, nct:] = mask_ref[0, 0]


def kernel(hidden_states, attention_mask, embeddings, position):
    B, S, D = hidden_states.shape
    _, NCT, _ = embeddings.shape
    pos = jnp.asarray(position, jnp.int32).reshape((1,))
    nd = D // D_BLK
    mask3 = attention_mask.reshape(B, 1, S)

    grid_spec = pltpu.PrefetchScalarGridSpec(
        num_scalar_prefetch=1,
        grid=(B, nd),
        in_specs=[
            pl.BlockSpec((1, S, D_BLK), lambda b, d, p: (b, 0, d)),
            pl.BlockSpec((1, 1, S), lambda b, d, p: (b, 0, 0)),
            pl.BlockSpec((1, NCT, D_BLK), lambda b, d, p: (p[0], 0, d)),
        ],
        out_specs=[
            pl.BlockSpec((1, NCT + S, D_BLK), lambda b, d, p: (b, 0, d)),
            pl.BlockSpec((1, 1, NCT + S), lambda b, d, p: (b, 0, 0)),
        ],
    )

    out_hid, out_mask = pl.pallas_call(
        _body,
        grid_spec=grid_spec,
        out_shape=[
            jax.ShapeDtypeStruct((B, NCT + S, D), hidden_states.dtype),
            jax.ShapeDtypeStruct((B, 1, NCT + S), attention_mask.dtype),
        ],
    )(pos, hidden_states, mask3, embeddings)
    return (out_hid, out_mask.reshape(B, NCT + S))


# speedup vs baseline: 2.1512x; 1.0264x over previous
import jax
import jax.numpy as jnp
from jax.experimental import pallas as pl
from jax.experimental.pallas import tpu as pltpu

D_BLK = 1024


def _body(pos_ref, hid_ref, mask_ref, emb_ref, out_ref, mask_out_ref):
    nct = emb_ref.shape[1]
    out_ref[0, :nct, :] = emb_ref[0]
    out_ref[0, nct:, :] = hid_ref[0]
    d = pl.program_id(1)

    @pl.when(d == 0)
    def _():
        mask_out_ref[0, 0, :nct] = jnp.ones((nct,), mask_out_ref.dtype)
        mask_out_ref[0, 0, nct:] = mask_ref[0, 0]


def kernel(hidden_states, attention_mask, embeddings, position):
    B, S, D = hidden_states.shape
    _, NCT, _ = embeddings.shape
    pos = jnp.asarray(position, jnp.int32).reshape((1,))
    nd = D // D_BLK
    mask3 = attention_mask.reshape(B, 1, S)

    grid_spec = pltpu.PrefetchScalarGridSpec(
        num_scalar_prefetch=1,
        grid=(B, nd),
        in_specs=[
            pl.BlockSpec((1, S, D_BLK), lambda b, d, p: (b, 0, d)),
            pl.BlockSpec((1, 1, S), lambda b, d, p: (b, 0, 0)),
            pl.BlockSpec((1, NCT, D_BLK), lambda b, d, p: (p[0], 0, d)),
        ],
        out_specs=[
            pl.BlockSpec((1, NCT + S, D_BLK), lambda b, d, p: (b, 0, d)),
            pl.BlockSpec((1, 1, NCT + S), lambda b, d, p: (b, 0, 0)),
        ],
    )

    out_hid, out_mask = pl.pallas_call(
        _body,
        grid_spec=grid_spec,
        out_shape=[
            jax.ShapeDtypeStruct((B, NCT + S, D), hidden_states.dtype),
            jax.ShapeDtypeStruct((B, 1, NCT + S), attention_mask.dtype),
        ],
    )(pos, hidden_states, mask3, embeddings)
    return (out_hid, out_mask.reshape(B, NCT + S))
